# Initial kernel scaffold; baseline (speedup 1.0000x reference)
import jax
import jax.numpy as jnp
from jax.experimental import pallas as pl


def kernel(feat, biases, all_h1s, all_w1s, all_h2s, all_w2s):
    raise NotImplementedError("write your pallas kernel here")


# speedup vs baseline: 95.4189x; 95.4189x over previous
import jax
import jax.numpy as jnp
from jax.experimental import pallas as pl
from jax.experimental.pallas import tpu as pltpu

R = 8
K = 2 * R + 1


def _pos_bias_kernel(biases_ref, out_ref, c_ref):
    h1 = pl.program_id(0)

    @pl.when(h1 == 0)
    def _():
        H = out_ref.shape[1]
        iw = jax.lax.broadcasted_iota(jnp.int32, (K, H, H), 1)
        ib = jax.lax.broadcasted_iota(jnp.int32, (K, H, H), 2)
        kw = ib - iw + R
        acc = jnp.zeros((K, H, H), jnp.float32)
        for kw0 in range(K):
            acc = acc + jnp.where(kw == kw0, biases_ref[:, kw0][:, None, None], 0.0)
        c_ref[...] = acc

    out_ref[...] = jnp.zeros_like(out_ref)
    for kh in range(K):
        a = h1 + kh - R

        @pl.when((a >= 0) & (a < out_ref.shape[2]))
        def _():
            out_ref[0, :, pl.ds(a, 1), :] = c_ref[kh][:, None, :]


def kernel(feat, biases, all_h1s, all_w1s, all_h2s, all_w2s):
    H, W = feat.shape[-2], feat.shape[-1]
    out = pl.pallas_call(
        _pos_bias_kernel,
        grid=(H,),
        in_specs=[pl.BlockSpec((K, K), lambda i: (0, 0))],
        out_specs=pl.BlockSpec((1, W, H, W), lambda i: (i, 0, 0, 0)),
        out_shape=jax.ShapeDtypeStruct((H, W, H, W), jnp.float32),
        scratch_shapes=[pltpu.VMEM((K, W, W), jnp.float32)],
    )(biases.astype(jnp.float32))
    return out[None, None]


# speedup vs baseline: 113.4402x; 1.1889x over previous
import jax
import jax.numpy as jnp
from jax.experimental import pallas as pl
from jax.experimental.pallas import tpu as pltpu

R = 8
K = 2 * R + 1
NSEM = 8


def _pos_bias_kernel(biases_ref, out_ref, cbig_ref, sems):
    h1 = pl.program_id(0)
    H = out_ref.shape[0]
    W = out_ref.shape[1]
    AP = cbig_ref.shape[1]

    @pl.when(h1 == 0)
    def _():
        cbig_ref[...] = jnp.zeros((W, AP, W), jnp.float32)
        iw = jax.lax.broadcasted_iota(jnp.int32, (W, W), 0)
        ib = jax.lax.broadcasted_iota(jnp.int32, (W, W), 1)
        kw = ib - iw + R
        for k in range(K):
            row = jnp.zeros((W, W), jnp.float32)
            for kw0 in range(K):
                row = row + jnp.where(kw == kw0, biases_ref[k, kw0], 0.0)
            cbig_ref[:, (H - 1) - R + k, :] = row

    sem_slot = jax.lax.rem(h1, NSEM)

    @pl.when(h1 >= NSEM)
    def _():
        pltpu.make_async_copy(
            cbig_ref.at[:, pl.ds(0, H), :], out_ref.at[0], sems.at[sem_slot]
        ).wait()

    pltpu.make_async_copy(
        cbig_ref.at[:, pl.ds((H - 1) - h1, H), :],
        out_ref.at[h1],
        sems.at[sem_slot],
    ).start()

    @pl.when(h1 == H - 1)
    def _():
        for j in range(NSEM):
            pltpu.make_async_copy(
                cbig_ref.at[:, pl.ds(0, H), :], out_ref.at[0], sems.at[j]
            ).wait()


def kernel(feat, biases, all_h1s, all_w1s, all_h2s, all_w2s):
    H, W = feat.shape[-2], feat.shape[-1]
    out = pl.pallas_call(
        _pos_bias_kernel,
        grid=(H,),
        in_specs=[pl.BlockSpec((K, K), lambda i: (0, 0))],
        out_specs=pl.BlockSpec(memory_space=pl.ANY),
        out_shape=jax.ShapeDtypeStruct((H, W, H, W), jnp.float32),
        scratch_shapes=[
            pltpu.VMEM((W, 2 * H, W), jnp.float32),
            pltpu.SemaphoreType.DMA((NSEM,)),
        ],
    )(biases.astype(jnp.float32))
    return out[None, None]


# speedup vs baseline: 113.5174x; 1.0007x over previous
import jax
import jax.numpy as jnp
from jax.experimental import pallas as pl
from jax.experimental.pallas import tpu as pltpu

R = 8
K = 2 * R + 1
NSEM = 8


def _pos_bias_kernel(biases_ref, out_ref, clo_ref, chi_ref, sems):
    h1 = pl.program_id(0)
    H = out_ref.shape[0]
    W = out_ref.shape[1]
    AP = clo_ref.shape[1]
    WH = W // 2

    @pl.when(h1 == 0)
    def _():
        clo_ref[...] = jnp.zeros((WH, AP, W), jnp.float32)
        chi_ref[...] = jnp.zeros((WH, AP, W), jnp.float32)
        iw = jax.lax.broadcasted_iota(jnp.int32, (W, W), 0)
        ib = jax.lax.broadcasted_iota(jnp.int32, (W, W), 1)
        kw = ib - iw + R
        for k in range(K):
            row = jnp.zeros((W, W), jnp.float32)
            for kw0 in range(K):
                row = row + jnp.where(kw == kw0, biases_ref[k, kw0], 0.0)
            clo_ref[:, (H - 1) - R + k, :] = row[:WH]
            chi_ref[:, (H - 1) - R + k, :] = row[WH:]

    sem_slot = jax.lax.rem(2 * h1, NSEM)
    sem_slot2 = jax.lax.rem(2 * h1 + 1, NSEM)

    @pl.when(2 * h1 >= NSEM)
    def _():
        pltpu.make_async_copy(
            clo_ref.at[:, pl.ds(0, H), :], out_ref.at[0, 0:WH], sems.at[sem_slot]
        ).wait()
        pltpu.make_async_copy(
            chi_ref.at[:, pl.ds(0, H), :], out_ref.at[0, WH:W], sems.at[sem_slot2]
        ).wait()

    pltpu.make_async_copy(
        clo_ref.at[:, pl.ds((H - 1) - h1, H), :],
        out_ref.at[h1, 0:WH],
        sems.at[sem_slot],
    ).start()
    pltpu.make_async_copy(
        chi_ref.at[:, pl.ds((H - 1) - h1, H), :],
        out_ref.at[h1, WH:W],
        sems.at[sem_slot2],
    ).start()

    @pl.when(h1 == H - 1)
    def _():
        for j in range(0, NSEM, 2):
            pltpu.make_async_copy(
                clo_ref.at[:, pl.ds(0, H), :], out_ref.at[0, 0:WH], sems.at[j]
            ).wait()
            pltpu.make_async_copy(
                chi_ref.at[:, pl.ds(0, H), :], out_ref.at[0, WH:W], sems.at[j + 1]
            ).wait()


def kernel(feat, biases, all_h1s, all_w1s, all_h2s, all_w2s):
    H, W = feat.shape[-2], feat.shape[-1]
    out = pl.pallas_call(
        _pos_bias_kernel,
        grid=(H,),
        in_specs=[pl.BlockSpec((K, K), lambda i: (0, 0))],
        out_specs=pl.BlockSpec(memory_space=pl.ANY),
        out_shape=jax.ShapeDtypeStruct((H, W, H, W), jnp.float32),
        scratch_shapes=[
            pltpu.VMEM((W // 2, 2 * H, W), jnp.float32),
            pltpu.VMEM((W // 2, 2 * H, W), jnp.float32),
            pltpu.SemaphoreType.DMA((NSEM,)),
        ],
    )(biases.astype(jnp.float32))
    return out[None, None]
